# baseline (speedup 1.0000x reference)
import functools

import jax
import jax.numpy as jnp
import numpy as np
from jax import lax
from jax.experimental import pallas as pl
from jax.experimental.pallas import tpu as pltpu
from jax.experimental.pallas import tpu_sc as plsc

B = 4096
NUM_FIELDS = 26
VOCAB = 100000
EMB = 64
NUM_DENSE = 13
NV = NUM_FIELDS + 1
INTER = NV * (NV - 1) // 2
TABLE_ROWS = NUM_FIELDS * VOCAB

NC, NS = 2, 16
NW = NC * NS
TOTAL_ROWS = B * NUM_FIELDS
ROWS_PER_W = TOTAL_ROWS // NW
CHUNK = 128
NCHUNK = ROWS_PER_W // CHUNK

PACK_BV = 32768
PACK_GRID = (TABLE_ROWS + PACK_BV - 1) // PACK_BV
QUAD_ROWS = PACK_GRID * (PACK_BV // 4)

_tril_i, _tril_j = np.tril_indices(NV, -1)
_pair_map = np.full((NV * NV,), INTER, dtype=np.int32)
_pair_map[_tril_i * NV + _tril_j] = np.arange(INTER, dtype=np.int32)


def _pack_body(in_ref, out_ref):
  t = in_ref[...]
  half = PACK_BV // 2
  ta = t[:, :half].T.astype(jnp.bfloat16)
  tb = t[:, half:].T.astype(jnp.bfloat16)
  cat = jnp.concatenate([ta, tb], axis=1)
  out_ref[...] = pltpu.bitcast(cat, jnp.int32)


def _pack_table(emb_table):
  et = emb_table.T
  return pl.pallas_call(
      _pack_body,
      grid=(PACK_GRID,),
      in_specs=[pl.BlockSpec((EMB, PACK_BV), lambda i: (0, i))],
      out_specs=pl.BlockSpec((PACK_BV // 4, 2 * EMB), lambda i: (i, 0)),
      out_shape=jax.ShapeDtypeStruct((QUAD_ROWS, 2 * EMB), jnp.int32),
  )(et)


def _sc_gather(table_p, idx3):
  mesh = plsc.VectorSubcoreMesh(core_axis_name="c", subcore_axis_name="s")

  @functools.partial(
      pl.kernel,
      mesh=mesh,
      out_type=jax.ShapeDtypeStruct((TOTAL_ROWS, 2 * EMB), jnp.int32),
      scratch_types=[
          pltpu.VMEM((NCHUNK, CHUNK), jnp.int32),
          pltpu.VMEM((CHUNK, 2 * EMB), jnp.int32),
          pltpu.SemaphoreType.DMA,
      ],
      compiler_params=pltpu.CompilerParams(use_tc_tiling_on_sc=False),
  )
  def k(table_hbm, idx_hbm, out_hbm, idx_v, rows_v, sem):
    wid = lax.axis_index("s") * NC + lax.axis_index("c")
    base = wid * ROWS_PER_W
    pltpu.sync_copy(idx_hbm.at[wid], idx_v)

    def body(c, _):
      pltpu.async_copy(table_hbm.at[idx_v.at[c]], rows_v, sem).wait()
      pltpu.sync_copy(rows_v, out_hbm.at[pl.ds(base + c * CHUNK, CHUNK)])
      return 0

    lax.fori_loop(0, NCHUNK, body, 0)

  return k(table_p, idx3)


def _tc_body(num_ref, emb_ref, sel_ref, bw0, bb0, bw1, bb1, bw2, bb2,
             tw0a, wz, tb0, tw1, tb1, tw2, tb2, tw3, tb3, tw4, tb4,
             out_ref):
  f32 = jnp.float32
  x = num_ref[...]
  h = jnp.maximum(jnp.dot(x, bw0[...], preferred_element_type=f32) + bb0[...], 0.0)
  h = jnp.maximum(jnp.dot(h, bw1[...], preferred_element_type=f32) + bb1[...], 0.0)
  bmo = jnp.maximum(jnp.dot(h, bw2[...], preferred_element_type=f32) + bb2[...], 0.0)
  ep = emb_ref[...].reshape(x.shape[0], NUM_FIELDS, 2 * EMB)
  s = sel_ref[...]
  word = jnp.where(s >= 1.5, ep[:, :, EMB:], ep[:, :, :EMB])
  odd = jnp.logical_or(s == 1.0, s == 3.0)
  bits = jnp.where(odd, word & jnp.int32(-65536), word << 16)
  emb = lax.bitcast_convert_type(bits, f32)
  t3 = jnp.concatenate([bmo[:, None, :], emb], axis=1)
  z3 = lax.dot_general(t3, t3, (((2,), (2,)), ((0,), (0,))),
                       preferred_element_type=f32)
  zf = z3.reshape(z3.shape[0], NV * NV)
  x1 = jnp.maximum(jnp.dot(bmo, tw0a[...], preferred_element_type=f32)
                   + jnp.dot(zf, wz[...], preferred_element_type=f32)
                   + tb0[...], 0.0)
  x2 = jnp.maximum(jnp.dot(x1, tw1[...], preferred_element_type=f32) + tb1[...], 0.0)
  x3 = jnp.maximum(jnp.dot(x2, tw2[...], preferred_element_type=f32) + tb2[...], 0.0)
  x4 = jnp.maximum(jnp.dot(x3, tw3[...], preferred_element_type=f32) + tb3[...], 0.0)
  out_ref[...] = jnp.dot(x4, tw4[...], preferred_element_type=f32) + tb4[...]


def kernel(numerical_input, categorical_inputs, emb_table,
           bw0, bb0, bw1, bb1, bw2, bb2,
           tw0, tb0, tw1, tb1, tw2, tb2, tw3, tb3, tw4, tb4):
  offsets = (jnp.arange(NUM_FIELDS, dtype=jnp.int32) * VOCAB)[None, :]
  idx = categorical_inputs + offsets
  half = PACK_BV // 2
  r = idx % PACK_BV
  prow = (idx // PACK_BV) * half + (r % half)
  hi_half = (r >= half).astype(jnp.int32)
  idx3 = (prow >> 1).reshape(NW, NCHUNK, CHUNK)
  sel = jnp.broadcast_to(
      (2 * hi_half + (prow & 1)).astype(jnp.bfloat16)[:, :, None],
      (B, NUM_FIELDS, EMB))
  tw0_pad = jnp.concatenate([tw0[EMB:], jnp.zeros((1, tw0.shape[1]), tw0.dtype)], axis=0)
  wz = jnp.take(tw0_pad, jnp.asarray(_pair_map), axis=0)
  tw0a = tw0[:EMB]

  table_p = _pack_table(emb_table)
  emb_rows = _sc_gather(table_p, idx3)

  bm = 256
  grid = (B // bm,)
  full = lambda shape: pl.BlockSpec(shape, lambda i: (0,) * len(shape))
  out = pl.pallas_call(
      _tc_body,
      grid=grid,
      in_specs=[
          pl.BlockSpec((bm, NUM_DENSE), lambda i: (i, 0)),
          pl.BlockSpec((bm * NUM_FIELDS, 2 * EMB), lambda i: (i, 0)),
          pl.BlockSpec((bm, NUM_FIELDS, EMB), lambda i: (i, 0, 0)),
          full((NUM_DENSE, 512)), full((1, 512)),
          full((512, 256)), full((1, 256)),
          full((256, EMB)), full((1, EMB)),
          full((EMB, 1024)), full((NV * NV, 1024)), full((1, 1024)),
          full((1024, 1024)), full((1, 1024)),
          full((1024, 512)), full((1, 512)),
          full((512, 256)), full((1, 256)),
          full((256, 1)), full((1, 1)),
      ],
      out_specs=pl.BlockSpec((bm, 1), lambda i: (i, 0)),
      out_shape=jax.ShapeDtypeStruct((B, 1), jnp.float32),
  )(
      numerical_input, emb_rows, sel,
      bw0, bb0[None, :], bw1, bb1[None, :], bw2, bb2[None, :],
      tw0a, wz, tb0[None, :],
      tw1, tb1[None, :], tw2, tb2[None, :], tw3, tb3[None, :],
      tw4, tb4[None, :],
  )
  return out


# speedup vs baseline: 1.1308x; 1.1308x over previous
import functools

import jax
import jax.numpy as jnp
import numpy as np
from jax import lax
from jax.experimental import pallas as pl
from jax.experimental.pallas import tpu as pltpu
from jax.experimental.pallas import tpu_sc as plsc

B = 4096
NUM_FIELDS = 26
VOCAB = 100000
EMB = 64
NUM_DENSE = 13
NV = NUM_FIELDS + 1
INTER = NV * (NV - 1) // 2
TABLE_ROWS = NUM_FIELDS * VOCAB

NC, NS = 2, 16
NW = NC * NS
TOTAL_ROWS = B * NUM_FIELDS
ROWS_PER_W = TOTAL_ROWS // NW
CHUNK = 128
NCHUNK = ROWS_PER_W // CHUNK

PACK_BV = 32768
PACK_GRID = (TABLE_ROWS + PACK_BV - 1) // PACK_BV
QUAD_ROWS = PACK_GRID * (PACK_BV // 4)

_tril_i, _tril_j = np.tril_indices(NV, -1)
_pair_map = np.full((NV * NV,), INTER, dtype=np.int32)
_pair_map[_tril_i * NV + _tril_j] = np.arange(INTER, dtype=np.int32)


def _pack_body(in_ref, out_ref):
  t = in_ref[...]
  half = PACK_BV // 2
  ta = t[:, :half].T.astype(jnp.bfloat16)
  tb = t[:, half:].T.astype(jnp.bfloat16)
  cat = jnp.concatenate([ta, tb], axis=1)
  out_ref[...] = pltpu.bitcast(cat, jnp.int32)


def _pack_table(emb_table):
  et = emb_table.T
  return pl.pallas_call(
      _pack_body,
      grid=(PACK_GRID,),
      in_specs=[pl.BlockSpec((EMB, PACK_BV), lambda i: (0, i))],
      out_specs=pl.BlockSpec((PACK_BV // 4, 2 * EMB), lambda i: (i, 0)),
      out_shape=jax.ShapeDtypeStruct((QUAD_ROWS, 2 * EMB), jnp.int32),
  )(et)


def _sc_gather(table_p, idx3):
  mesh = plsc.VectorSubcoreMesh(core_axis_name="c", subcore_axis_name="s")

  @functools.partial(
      pl.kernel,
      mesh=mesh,
      out_type=jax.ShapeDtypeStruct((TOTAL_ROWS, 2 * EMB), jnp.int32),
      scratch_types=[
          pltpu.VMEM((NCHUNK, CHUNK), jnp.int32),
          pltpu.VMEM((CHUNK, 2 * EMB), jnp.int32),
          pltpu.SemaphoreType.DMA,
      ],
      compiler_params=pltpu.CompilerParams(use_tc_tiling_on_sc=False),
  )
  def k(table_hbm, idx_hbm, out_hbm, idx_v, rows_v, sem):
    wid = lax.axis_index("s") * NC + lax.axis_index("c")
    base = wid * ROWS_PER_W
    pltpu.sync_copy(idx_hbm.at[wid], idx_v)

    def body(c, _):
      pltpu.async_copy(table_hbm.at[idx_v.at[c]], rows_v, sem).wait()
      pltpu.sync_copy(rows_v, out_hbm.at[pl.ds(base + c * CHUNK, CHUNK)])
      return 0

    lax.fori_loop(0, NCHUNK, body, 0)

  return k(table_p, idx3)


def _tc_body(num_ref, emb_ref, sel_ref, bw0, bb0, bw1, bb1, bw2, bb2,
             tw0a, wz, tb0, tw1, tb1, tw2, tb2, tw3, tb3, tw4, tb4,
             out_ref):
  f32 = jnp.float32
  x = num_ref[...]
  h = jnp.maximum(jnp.dot(x, bw0[...], preferred_element_type=f32) + bb0[...], 0.0)
  h = jnp.maximum(jnp.dot(h, bw1[...], preferred_element_type=f32) + bb1[...], 0.0)
  bmo = jnp.maximum(jnp.dot(h, bw2[...], preferred_element_type=f32) + bb2[...], 0.0)
  ep = emb_ref[...]
  s = sel_ref[...]
  word = jnp.where(s >= 1.5, ep[:, :, EMB:], ep[:, :, :EMB])
  odd = jnp.logical_or(s == 1.0, s == 3.0)
  bits = jnp.where(odd, word & jnp.int32(-65536), word << 16)
  emb = lax.bitcast_convert_type(bits, f32)
  t3 = jnp.concatenate([bmo[:, None, :], emb], axis=1)
  z3 = lax.dot_general(t3, t3, (((2,), (2,)), ((0,), (0,))),
                       preferred_element_type=f32)
  zf = z3.reshape(z3.shape[0], NV * NV)
  x1 = jnp.maximum(jnp.dot(bmo, tw0a[...], preferred_element_type=f32)
                   + jnp.dot(zf, wz[...], preferred_element_type=f32)
                   + tb0[...], 0.0)
  x2 = jnp.maximum(jnp.dot(x1, tw1[...], preferred_element_type=f32) + tb1[...], 0.0)
  x3 = jnp.maximum(jnp.dot(x2, tw2[...], preferred_element_type=f32) + tb2[...], 0.0)
  x4 = jnp.maximum(jnp.dot(x3, tw3[...], preferred_element_type=f32) + tb3[...], 0.0)
  out_ref[...] = jnp.dot(x4, tw4[...], preferred_element_type=f32) + tb4[...]


def kernel(numerical_input, categorical_inputs, emb_table,
           bw0, bb0, bw1, bb1, bw2, bb2,
           tw0, tb0, tw1, tb1, tw2, tb2, tw3, tb3, tw4, tb4):
  offsets = (jnp.arange(NUM_FIELDS, dtype=jnp.int32) * VOCAB)[None, :]
  idx = categorical_inputs + offsets
  half = PACK_BV // 2
  r = idx % PACK_BV
  prow = (idx // PACK_BV) * half + (r % half)
  hi_half = (r >= half).astype(jnp.int32)
  idx3 = (prow >> 1).reshape(NW, NCHUNK, CHUNK)
  sel = jnp.broadcast_to(
      (2 * hi_half + (prow & 1)).astype(jnp.bfloat16)[:, :, None],
      (B, NUM_FIELDS, EMB))
  tw0_pad = jnp.concatenate([tw0[EMB:], jnp.zeros((1, tw0.shape[1]), tw0.dtype)], axis=0)
  wz = jnp.take(tw0_pad, jnp.asarray(_pair_map), axis=0)
  tw0a = tw0[:EMB]

  table_p = _pack_table(emb_table)
  emb_rows = _sc_gather(table_p, idx3)
  emb3 = emb_rows.reshape(B, NUM_FIELDS, 2 * EMB)

  bm = 256
  grid = (B // bm,)
  full = lambda shape: pl.BlockSpec(shape, lambda i: (0,) * len(shape))
  out = pl.pallas_call(
      _tc_body,
      grid=grid,
      in_specs=[
          pl.BlockSpec((bm, NUM_DENSE), lambda i: (i, 0)),
          pl.BlockSpec((bm, NUM_FIELDS, 2 * EMB), lambda i: (i, 0, 0)),
          pl.BlockSpec((bm, NUM_FIELDS, EMB), lambda i: (i, 0, 0)),
          full((NUM_DENSE, 512)), full((1, 512)),
          full((512, 256)), full((1, 256)),
          full((256, EMB)), full((1, EMB)),
          full((EMB, 1024)), full((NV * NV, 1024)), full((1, 1024)),
          full((1024, 1024)), full((1, 1024)),
          full((1024, 512)), full((1, 512)),
          full((512, 256)), full((1, 256)),
          full((256, 1)), full((1, 1)),
      ],
      out_specs=pl.BlockSpec((bm, 1), lambda i: (i, 0)),
      out_shape=jax.ShapeDtypeStruct((B, 1), jnp.float32),
  )(
      numerical_input, emb3, sel,
      bw0, bb0[None, :], bw1, bb1[None, :], bw2, bb2[None, :],
      tw0a, wz, tb0[None, :],
      tw1, tb1[None, :], tw2, tb2[None, :], tw3, tb3[None, :],
      tw4, tb4[None, :],
  )
  return out


# speedup vs baseline: 1.2217x; 1.0803x over previous
import functools

import jax
import jax.numpy as jnp
import numpy as np
from jax import lax
from jax.experimental import pallas as pl
from jax.experimental.pallas import tpu as pltpu
from jax.experimental.pallas import tpu_sc as plsc

B = 4096
NUM_FIELDS = 26
VOCAB = 100000
EMB = 64
NUM_DENSE = 13
NV = NUM_FIELDS + 1
INTER = NV * (NV - 1) // 2
TABLE_ROWS = NUM_FIELDS * VOCAB

NC, NS = 2, 16
NW = NC * NS
TOTAL_ROWS = B * NUM_FIELDS
ROWS_PER_W = TOTAL_ROWS // NW
CHUNK = 128
NCHUNK = ROWS_PER_W // CHUNK

PACK_BV = 32768
PACK_GRID = (TABLE_ROWS + PACK_BV - 1) // PACK_BV
QUAD_ROWS = PACK_GRID * (PACK_BV // 4)

_tril_i, _tril_j = np.tril_indices(NV, -1)
_pair_map = np.full((NV * NV,), INTER, dtype=np.int32)
_pair_map[_tril_i * NV + _tril_j] = np.arange(INTER, dtype=np.int32)


def _pack_body(in_ref, out_ref):
  t = in_ref[...]
  half = PACK_BV // 2
  ta = t[:, :half].T.astype(jnp.bfloat16)
  tb = t[:, half:].T.astype(jnp.bfloat16)
  cat = jnp.concatenate([ta, tb], axis=1)
  out_ref[...] = pltpu.bitcast(cat, jnp.int32)


def _pack_table(emb_table):
  et = emb_table.T
  return pl.pallas_call(
      _pack_body,
      grid=(PACK_GRID,),
      in_specs=[pl.BlockSpec((EMB, PACK_BV), lambda i: (0, i))],
      out_specs=pl.BlockSpec((PACK_BV // 4, 2 * EMB), lambda i: (i, 0)),
      out_shape=jax.ShapeDtypeStruct((QUAD_ROWS, 2 * EMB), jnp.int32),
  )(et)


def _sc_gather(table_p, idx3):
  mesh = plsc.VectorSubcoreMesh(core_axis_name="c", subcore_axis_name="s")

  @functools.partial(
      pl.kernel,
      mesh=mesh,
      out_type=jax.ShapeDtypeStruct((TOTAL_ROWS, 2 * EMB), jnp.int32),
      scratch_types=[
          pltpu.VMEM((NCHUNK, CHUNK), jnp.int32),
          pltpu.VMEM((CHUNK, 2 * EMB), jnp.int32),
          pltpu.SemaphoreType.DMA,
      ],
      compiler_params=pltpu.CompilerParams(use_tc_tiling_on_sc=False),
  )
  def k(table_hbm, idx_hbm, out_hbm, idx_v, rows_v, sem):
    wid = lax.axis_index("s") * NC + lax.axis_index("c")
    base = wid * ROWS_PER_W
    pltpu.sync_copy(idx_hbm.at[wid], idx_v)

    def body(c, _):
      pltpu.async_copy(table_hbm.at[idx_v.at[c]], rows_v, sem).wait()
      pltpu.sync_copy(rows_v, out_hbm.at[pl.ds(base + c * CHUNK, CHUNK)])
      return 0

    lax.fori_loop(0, NCHUNK, body, 0)

  return k(table_p, idx3)


def _tc_body(num_ref, emb_ref, sel_ref, bw0, bb0, bw1, bb1, bw2, bb2,
             tw0a, wz, tb0, tw1, tb1, tw2, tb2, tw3, tb3, tw4, tb4,
             out_ref):
  f32 = jnp.float32
  x = num_ref[...]
  h = jnp.maximum(jnp.dot(x, bw0[...], preferred_element_type=f32) + bb0[...], 0.0)
  h = jnp.maximum(jnp.dot(h, bw1[...], preferred_element_type=f32) + bb1[...], 0.0)
  bmo = jnp.maximum(jnp.dot(h, bw2[...], preferred_element_type=f32) + bb2[...], 0.0)
  ep = emb_ref[...]
  s = sel_ref[...]
  word = jnp.where(s >= 1.5, ep[:, :, EMB:], ep[:, :, :EMB])
  odd = jnp.logical_or(s == 1.0, s == 3.0)
  bits = jnp.where(odd, word & jnp.int32(-65536), word << 16)
  emb = lax.bitcast_convert_type(bits, f32)
  t3 = jnp.concatenate([bmo[:, None, :], emb], axis=1)
  z3 = lax.dot_general(t3, t3, (((2,), (2,)), ((0,), (0,))),
                       preferred_element_type=f32)
  zf = z3.reshape(z3.shape[0], NV * NV)
  x1 = jnp.maximum(jnp.dot(bmo, tw0a[...], preferred_element_type=f32)
                   + jnp.dot(zf, wz[...], preferred_element_type=f32)
                   + tb0[...], 0.0)
  x2 = jnp.maximum(jnp.dot(x1, tw1[...], preferred_element_type=f32) + tb1[...], 0.0)
  x3 = jnp.maximum(jnp.dot(x2, tw2[...], preferred_element_type=f32) + tb2[...], 0.0)
  x4 = jnp.maximum(jnp.dot(x3, tw3[...], preferred_element_type=f32) + tb3[...], 0.0)
  out_ref[...] = jnp.dot(x4, tw4[...], preferred_element_type=f32) + tb4[...]


def kernel(numerical_input, categorical_inputs, emb_table,
           bw0, bb0, bw1, bb1, bw2, bb2,
           tw0, tb0, tw1, tb1, tw2, tb2, tw3, tb3, tw4, tb4):
  offsets = (jnp.arange(NUM_FIELDS, dtype=jnp.int32) * VOCAB)[None, :]
  idx = categorical_inputs + offsets
  half = PACK_BV // 2
  r = idx % PACK_BV
  prow = (idx // PACK_BV) * half + (r % half)
  hi_half = (r >= half).astype(jnp.int32)
  idx3 = (prow >> 1).reshape(NW, NCHUNK, CHUNK)
  sel = jnp.broadcast_to(
      (2 * hi_half + (prow & 1)).astype(jnp.float32)[:, :, None],
      (B, NUM_FIELDS, EMB))
  tw0_pad = jnp.concatenate([tw0[EMB:], jnp.zeros((1, tw0.shape[1]), tw0.dtype)], axis=0)
  wz = jnp.take(tw0_pad, jnp.asarray(_pair_map), axis=0)
  tw0a = tw0[:EMB]

  table_p = _pack_table(emb_table)
  emb_rows = _sc_gather(table_p, idx3)
  emb3 = emb_rows.reshape(B, NUM_FIELDS, 2 * EMB)

  bm = 256
  grid = (B // bm,)
  full = lambda shape: pl.BlockSpec(shape, lambda i: (0,) * len(shape))
  out = pl.pallas_call(
      _tc_body,
      grid=grid,
      in_specs=[
          pl.BlockSpec((bm, NUM_DENSE), lambda i: (i, 0)),
          pl.BlockSpec((bm, NUM_FIELDS, 2 * EMB), lambda i: (i, 0, 0)),
          pl.BlockSpec((bm, NUM_FIELDS, EMB), lambda i: (i, 0, 0)),
          full((NUM_DENSE, 512)), full((1, 512)),
          full((512, 256)), full((1, 256)),
          full((256, EMB)), full((1, EMB)),
          full((EMB, 1024)), full((NV * NV, 1024)), full((1, 1024)),
          full((1024, 1024)), full((1, 1024)),
          full((1024, 512)), full((1, 512)),
          full((512, 256)), full((1, 256)),
          full((256, 1)), full((1, 1)),
      ],
      out_specs=pl.BlockSpec((bm, 1), lambda i: (i, 0)),
      out_shape=jax.ShapeDtypeStruct((B, 1), jnp.float32),
  )(
      numerical_input, emb3, sel,
      bw0, bb0[None, :], bw1, bb1[None, :], bw2, bb2[None, :],
      tw0a, wz, tb0[None, :],
      tw1, tb1[None, :], tw2, tb2[None, :], tw3, tb3[None, :],
      tw4, tb4[None, :],
  )
  return out


# speedup vs baseline: 1.2594x; 1.0309x over previous
import functools

import jax
import jax.numpy as jnp
import numpy as np
from jax import lax
from jax.experimental import pallas as pl
from jax.experimental.pallas import tpu as pltpu
from jax.experimental.pallas import tpu_sc as plsc

B = 4096
NUM_FIELDS = 26
VOCAB = 100000
EMB = 64
NUM_DENSE = 13
NV = NUM_FIELDS + 1
INTER = NV * (NV - 1) // 2
TABLE_ROWS = NUM_FIELDS * VOCAB

NC, NS = 2, 16
NW = NC * NS
TOTAL_ROWS = B * NUM_FIELDS
ROWS_PER_W = TOTAL_ROWS // NW
CHUNK = 128
NCHUNK = ROWS_PER_W // CHUNK

PACK_BV = 32768
PACK_GRID = (TABLE_ROWS + PACK_BV - 1) // PACK_BV
QUAD_ROWS = PACK_GRID * (PACK_BV // 4)

_tril_i, _tril_j = np.tril_indices(NV, -1)
_pair_map = np.full((NV * NV,), INTER, dtype=np.int32)
_pair_map[_tril_i * NV + _tril_j] = np.arange(INTER, dtype=np.int32)


def _pack_body(in_ref, out_ref):
  t = in_ref[...]
  half = PACK_BV // 2
  ta = t[:, :half].T.astype(jnp.bfloat16)
  tb = t[:, half:].T.astype(jnp.bfloat16)
  cat = jnp.concatenate([ta, tb], axis=1)
  out_ref[...] = pltpu.bitcast(cat, jnp.int32)


def _pack_table(emb_table):
  et = emb_table.T
  return pl.pallas_call(
      _pack_body,
      grid=(PACK_GRID,),
      in_specs=[pl.BlockSpec((EMB, PACK_BV), lambda i: (0, i))],
      out_specs=pl.BlockSpec((PACK_BV // 4, 2 * EMB), lambda i: (i, 0)),
      out_shape=jax.ShapeDtypeStruct((QUAD_ROWS, 2 * EMB), jnp.int32),
  )(et)


def _sc_gather(table_p, idx3):
  mesh = plsc.VectorSubcoreMesh(core_axis_name="c", subcore_axis_name="s")

  @functools.partial(
      pl.kernel,
      mesh=mesh,
      out_type=jax.ShapeDtypeStruct((TOTAL_ROWS, 2 * EMB), jnp.int32),
      scratch_types=[
          pltpu.VMEM((NCHUNK, CHUNK), jnp.int32),
          pltpu.VMEM((CHUNK, 2 * EMB), jnp.int32),
          pltpu.VMEM((CHUNK, 2 * EMB), jnp.int32),
          pltpu.SemaphoreType.DMA,
          pltpu.SemaphoreType.DMA,
      ],
      compiler_params=pltpu.CompilerParams(use_tc_tiling_on_sc=False),
  )
  def k(table_hbm, idx_hbm, out_hbm, idx_v, rows_a, rows_b, sem_a, sem_b):
    wid = lax.axis_index("s") * NC + lax.axis_index("c")
    base = wid * ROWS_PER_W
    pltpu.sync_copy(idx_hbm.at[wid], idx_v)

    def fire(c, rows, sem):
      pltpu.async_copy(table_hbm.at[idx_v.at[c]], rows, sem)

    def drain(c, rows, sem):
      pltpu.make_async_copy(table_hbm.at[idx_v.at[c]], rows, sem).wait()

    def out_slice(c):
      return out_hbm.at[pl.ds(base + c * CHUNK, CHUNK)]

    fire(0, rows_a, sem_a)

    def body(i, _):
      c = 2 * i
      fire(c + 1, rows_b, sem_b)
      drain(c, rows_a, sem_a)
      pltpu.sync_copy(rows_a, out_slice(c))

      @pl.when(c + 2 < NCHUNK)
      def _():
        fire(c + 2, rows_a, sem_a)

      drain(c + 1, rows_b, sem_b)
      pltpu.sync_copy(rows_b, out_slice(c + 1))
      return 0

    lax.fori_loop(0, NCHUNK // 2, body, 0)

  return k(table_p, idx3)


def _tc_body(num_ref, emb_ref, sel_ref, bw0, bb0, bw1, bb1, bw2, bb2,
             tw0a, wz, tb0, tw1, tb1, tw2, tb2, tw3, tb3, tw4, tb4,
             out_ref):
  f32 = jnp.float32
  x = num_ref[...]
  h = jnp.maximum(jnp.dot(x, bw0[...], preferred_element_type=f32) + bb0[...], 0.0)
  h = jnp.maximum(jnp.dot(h, bw1[...], preferred_element_type=f32) + bb1[...], 0.0)
  bmo = jnp.maximum(jnp.dot(h, bw2[...], preferred_element_type=f32) + bb2[...], 0.0)
  ep = emb_ref[...]
  s = sel_ref[...]
  word = jnp.where(s >= 1.5, ep[:, :, EMB:], ep[:, :, :EMB])
  odd = jnp.logical_or(s == 1.0, s == 3.0)
  bits = jnp.where(odd, word & jnp.int32(-65536), word << 16)
  emb = lax.bitcast_convert_type(bits, f32)
  t3 = jnp.concatenate([bmo[:, None, :], emb], axis=1)
  z3 = lax.dot_general(t3, t3, (((2,), (2,)), ((0,), (0,))),
                       preferred_element_type=f32)
  zf = z3.reshape(z3.shape[0], NV * NV)
  x1 = jnp.maximum(jnp.dot(bmo, tw0a[...], preferred_element_type=f32)
                   + jnp.dot(zf, wz[...], preferred_element_type=f32)
                   + tb0[...], 0.0)
  x2 = jnp.maximum(jnp.dot(x1, tw1[...], preferred_element_type=f32) + tb1[...], 0.0)
  x3 = jnp.maximum(jnp.dot(x2, tw2[...], preferred_element_type=f32) + tb2[...], 0.0)
  x4 = jnp.maximum(jnp.dot(x3, tw3[...], preferred_element_type=f32) + tb3[...], 0.0)
  out_ref[...] = jnp.dot(x4, tw4[...], preferred_element_type=f32) + tb4[...]


def kernel(numerical_input, categorical_inputs, emb_table,
           bw0, bb0, bw1, bb1, bw2, bb2,
           tw0, tb0, tw1, tb1, tw2, tb2, tw3, tb3, tw4, tb4):
  offsets = (jnp.arange(NUM_FIELDS, dtype=jnp.int32) * VOCAB)[None, :]
  idx = categorical_inputs + offsets
  half = PACK_BV // 2
  r = idx % PACK_BV
  prow = (idx // PACK_BV) * half + (r % half)
  hi_half = (r >= half).astype(jnp.int32)
  idx3 = (prow >> 1).reshape(NW, NCHUNK, CHUNK)
  sel = jnp.broadcast_to(
      (2 * hi_half + (prow & 1)).astype(jnp.float32)[:, :, None],
      (B, NUM_FIELDS, EMB))
  tw0_pad = jnp.concatenate([tw0[EMB:], jnp.zeros((1, tw0.shape[1]), tw0.dtype)], axis=0)
  wz = jnp.take(tw0_pad, jnp.asarray(_pair_map), axis=0)
  tw0a = tw0[:EMB]

  table_p = _pack_table(emb_table)
  emb_rows = _sc_gather(table_p, idx3)
  emb3 = emb_rows.reshape(B, NUM_FIELDS, 2 * EMB)

  bm = 256
  grid = (B // bm,)
  full = lambda shape: pl.BlockSpec(shape, lambda i: (0,) * len(shape))
  out = pl.pallas_call(
      _tc_body,
      grid=grid,
      in_specs=[
          pl.BlockSpec((bm, NUM_DENSE), lambda i: (i, 0)),
          pl.BlockSpec((bm, NUM_FIELDS, 2 * EMB), lambda i: (i, 0, 0)),
          pl.BlockSpec((bm, NUM_FIELDS, EMB), lambda i: (i, 0, 0)),
          full((NUM_DENSE, 512)), full((1, 512)),
          full((512, 256)), full((1, 256)),
          full((256, EMB)), full((1, EMB)),
          full((EMB, 1024)), full((NV * NV, 1024)), full((1, 1024)),
          full((1024, 1024)), full((1, 1024)),
          full((1024, 512)), full((1, 512)),
          full((512, 256)), full((1, 256)),
          full((256, 1)), full((1, 1)),
      ],
      out_specs=pl.BlockSpec((bm, 1), lambda i: (i, 0)),
      out_shape=jax.ShapeDtypeStruct((B, 1), jnp.float32),
  )(
      numerical_input, emb3, sel,
      bw0, bb0[None, :], bw1, bb1[None, :], bw2, bb2[None, :],
      tw0a, wz, tb0[None, :],
      tw1, tb1[None, :], tw2, tb2[None, :], tw3, tb3[None, :],
      tw4, tb4[None, :],
  )
  return out
